# baseline (speedup 1.0000x reference)
import jax
import jax.numpy as jnp
from jax import lax
from jax.experimental import pallas as pl
from jax.experimental.pallas import tpu as pltpu
from jax.experimental.pallas import tpu_sc as plsc

B = 16384
HIST = 50
D = 16
V = 1000000
NC = 2
NS = 16
NW = NC * NS
N_PER_W = B // NW
C = 64
NCHUNK = N_PER_W // C
SROWS = C * HIST
SIDX_ROWS = SROWS // 128

SB = 512
NSB = V // SB // NW * NW + 1
KMAX = 61
TAIL = V - NSB * SB


def _transpose_body(xiT, xuT, tailI, tailU, itemRM, userRM,
                    tiI, tiU, rowI, rowU, semLI, semLU, semS):
    wid = lax.axis_index("s") * NC + lax.axis_index("c")
    lane = lax.iota(jnp.int32, 16)

    def fire_loads(k, soff):
        off = pl.multiple_of((wid + k * NW) * SB, 128)
        pltpu.async_copy(xiT.at[pl.ds(0, 8), pl.ds(off, SB)],
                         tiI.at[pl.ds(0, 8), pl.ds(soff, SB)], semLI)
        pltpu.async_copy(xiT.at[pl.ds(8, 8), pl.ds(off, SB)],
                         tiI.at[pl.ds(8, 8), pl.ds(soff, SB)], semLI)
        pltpu.async_copy(xuT.at[pl.ds(0, 8), pl.ds(off, SB)],
                         tiU.at[pl.ds(0, 8), pl.ds(soff, SB)], semLU)
        pltpu.async_copy(xuT.at[pl.ds(8, 8), pl.ds(off, SB)],
                         tiU.at[pl.ds(8, 8), pl.ds(soff, SB)], semLU)

    def wait_loads(sem, ti):
        for h in (0, 8):
            pltpu.make_async_copy(xiT.at[pl.ds(0, 8), pl.ds(0, SB)],
                                  ti.at[pl.ds(h, 8), pl.ds(0, SB)], sem).wait()

    tvecs = [(lane + j) & 15 for j in range(16)]
    svecs = [t * D + lane for t in tvecs]

    def transpose_block(ti, soff, row, roff):
        def colgrp(g, _):
            cbase = soff + g * 16
            obase = roff + g * 256
            for j in range(16):
                vec = plsc.load_gather(ti, [lane, cbase + tvecs[j]])
                plsc.store_scatter(row, [obase + svecs[j]], vec)
            return 0
        lax.fori_loop(0, SB // 16, colgrp, 0)

    def drain_store(row, roff, dst):
        pltpu.make_async_copy(row.at[pl.ds(roff, SB * D)],
                              dst.at[pl.ds(0, SB * D)], semS).wait()

    def body(k, _):
        s = k % 2
        soff = pl.multiple_of(s * SB, 128)
        roff = pl.multiple_of(s * SB * D, 8)
        off = pl.multiple_of((wid + k * NW) * SB * D, 8)

        @pl.when(k >= 2)
        def _():
            drain_store(rowI, roff, itemRM)
            drain_store(rowU, roff, userRM)

        @pl.when(k + 1 < KMAX)
        def _():
            fire_loads(k + 1, pl.multiple_of((1 - s) * SB, 128))

        wait_loads(semLI, tiI)
        transpose_block(tiI, soff, rowI, roff)
        pltpu.async_copy(rowI.at[pl.ds(roff, SB * D)],
                         itemRM.at[pl.ds(off, SB * D)], semS)

        wait_loads(semLU, tiU)
        transpose_block(tiU, soff, rowU, roff)
        pltpu.async_copy(rowU.at[pl.ds(roff, SB * D)],
                         userRM.at[pl.ds(off, SB * D)], semS)
        return 0

    fire_loads(0, 0)
    lax.fori_loop(0, KMAX, body, 0)
    for s in (0, 1):
        drain_store(rowI, s * SB * D, itemRM)
        drain_store(rowU, s * SB * D, userRM)

    @pl.when(wid == 0)
    def _():
        off = pl.multiple_of((NSB - 1) * SB, 128)
        pltpu.sync_copy(xiT.at[pl.ds(0, 8), pl.ds(off, SB)],
                        tiI.at[pl.ds(0, 8), pl.ds(0, SB)])
        pltpu.sync_copy(xiT.at[pl.ds(8, 8), pl.ds(off, SB)],
                        tiI.at[pl.ds(8, 8), pl.ds(0, SB)])
        transpose_block(tiI, 0, rowI, 0)
        pltpu.sync_copy(rowI.at[pl.ds(0, SB * D)],
                        itemRM.at[pl.ds((NSB - 1) * SB * D, SB * D)])
        pltpu.sync_copy(tailI, itemRM.at[pl.ds(NSB * SB * D, TAIL * D)])

    @pl.when(wid == NW - 1)
    def _():
        off = pl.multiple_of((NSB - 1) * SB, 128)
        pltpu.sync_copy(xuT.at[pl.ds(0, 8), pl.ds(off, SB)],
                        tiU.at[pl.ds(0, 8), pl.ds(0, SB)])
        pltpu.sync_copy(xuT.at[pl.ds(8, 8), pl.ds(off, SB)],
                        tiU.at[pl.ds(8, 8), pl.ds(0, SB)])
        transpose_block(tiU, 0, rowU, 0)
        pltpu.sync_copy(rowU.at[pl.ds(0, SB * D)],
                        userRM.at[pl.ds((NSB - 1) * SB * D, SB * D)])
        pltpu.sync_copy(tailU, userRM.at[pl.ds(NSB * SB * D, TAIL * D)])


def _gather_body(seqs_hbm, users_hbm, pos_hbm, neg_hbm, eu_hbm, ei_hbm,
                 out_hbm, sidx, srows, uidx, pidx, nidx, urows, prows, nrows,
                 psc, nsc, pt, nt, sem0, sem1):
    wid = lax.axis_index("s") * NC + lax.axis_index("c")
    base = wid * N_PER_W
    sems = (sem0, sem1)
    lane = lax.iota(jnp.int32, 16)

    def fire(c):
        s = c % 2
        g = wid * NCHUNK + c
        pltpu.sync_copy(seqs_hbm.at[pl.ds(g * SROWS, SROWS)], sidx.at[s])
        pltpu.sync_copy(users_hbm.at[pl.ds(g * C, C)], uidx.at[s])
        pltpu.sync_copy(pos_hbm.at[pl.ds(g * C, C)], pidx.at[s])
        pltpu.sync_copy(neg_hbm.at[pl.ds(g * C, C)], nidx.at[s])
        cps = []
        for j in range(SIDX_ROWS):
            cps.append(pltpu.async_copy(
                ei_hbm.at[sidx.at[s].at[pl.ds(j * 128, 128)]],
                srows.at[s].at[pl.ds(j * 128, 128)], sems[s]))
        cps.append(pltpu.async_copy(eu_hbm.at[uidx.at[s]], urows.at[s], sems[s]))
        cps.append(pltpu.async_copy(ei_hbm.at[pidx.at[s]], prows.at[s], sems[s]))
        cps.append(pltpu.async_copy(ei_hbm.at[nidx.at[s]], nrows.at[s], sems[s]))
        return cps

    def compute(c):
        s = c % 2
        srows_s, urows_s, prows_s, nrows_s = (
            srows.at[s], urows.at[s], prows.at[s], nrows.at[s])

        def group(g, _):
            def item(l, _):
                i = g * 16 + l
                ib = i * HIST
                a0 = srows_s[ib + 0, :]
                a1 = srows_s[ib + 1, :]
                a2 = srows_s[ib + 2, :]
                a3 = srows_s[ib + 3, :]
                for j in range(4, HIST, 4):
                    a0 = a0 + srows_s[ib + j + 0, :]
                    a1 = a1 + srows_s[ib + j + 1, :]
                    if j + 2 < HIST:
                        a2 = a2 + srows_s[ib + j + 2, :]
                        a3 = a3 + srows_s[ib + j + 3, :]
                acc = (a0 + a1) + (a2 + a3)
                h = urows_s[i, :] + acc * (1.0 / HIST)
                col = jnp.full((16,), l, jnp.int32)
                plsc.store_scatter(pt, [lane, col], h * prows_s[i, :])
                plsc.store_scatter(nt, [lane, col], h * nrows_s[i, :])
                return 0

            lax.fori_loop(0, 16, item, 0)
            pvec = pt[0, :]
            nvec = nt[0, :]
            for d in range(1, D):
                pvec = pvec + pt[d, :]
                nvec = nvec + nt[d, :]
            off = (c * C) + g * 16
            psc[pl.ds(off, 16)] = pvec
            nsc[pl.ds(off, 16)] = nvec
            return 0

        lax.fori_loop(0, C // 16, group, 0)

    pending = fire(0)
    for c in range(NCHUNK):
        nxt = fire(c + 1) if c + 1 < NCHUNK else []
        for cp in pending:
            cp.wait()
        compute(c)
        pending = nxt

    pltpu.sync_copy(psc, out_hbm.at[pl.ds(base, N_PER_W)])
    pltpu.sync_copy(nsc, out_hbm.at[pl.ds(B + base, N_PER_W)])


@jax.jit
def kernel(users, seqs, posItems, negItems, embedUser, embedItem):
    seqs_r = seqs.reshape(B * HIST)
    tailI = embedItem[NSB * SB:, :].reshape(TAIL * D)
    tailU = embedUser[NSB * SB:, :].reshape(TAIL * D)

    mesh = plsc.VectorSubcoreMesh(core_axis_name="c", subcore_axis_name="s")

    transpose = pl.kernel(
        _transpose_body,
        out_type=(jax.ShapeDtypeStruct((V * D,), jnp.float32),
                  jax.ShapeDtypeStruct((V * D,), jnp.float32)),
        mesh=mesh,
        compiler_params=pltpu.CompilerParams(
            needs_layout_passes=False, use_tc_tiling_on_sc=True),
        scratch_types=[
            pltpu.VMEM((16, 2 * SB), jnp.float32),
            pltpu.VMEM((16, 2 * SB), jnp.float32),
            pltpu.VMEM((2 * SB * D,), jnp.float32),
            pltpu.VMEM((2 * SB * D,), jnp.float32),
            pltpu.SemaphoreType.DMA,
            pltpu.SemaphoreType.DMA,
            pltpu.SemaphoreType.DMA,
        ],
    )
    itemRM, userRM = transpose(embedItem.T, embedUser.T, tailI, tailU)
    itemRM = itemRM.reshape(V, D)
    userRM = userRM.reshape(V, D)

    gather = pl.kernel(
        _gather_body,
        out_type=jax.ShapeDtypeStruct((2 * B,), jnp.float32),
        mesh=mesh,
        compiler_params=pltpu.CompilerParams(
            needs_layout_passes=False, use_tc_tiling_on_sc=False),
        scratch_types=[
            pltpu.VMEM((2, SROWS), jnp.int32),
            pltpu.VMEM((2, SROWS, D), jnp.float32),
            pltpu.VMEM((2, C), jnp.int32),
            pltpu.VMEM((2, C), jnp.int32),
            pltpu.VMEM((2, C), jnp.int32),
            pltpu.VMEM((2, C, D), jnp.float32),
            pltpu.VMEM((2, C, D), jnp.float32),
            pltpu.VMEM((2, C, D), jnp.float32),
            pltpu.VMEM((N_PER_W,), jnp.float32),
            pltpu.VMEM((N_PER_W,), jnp.float32),
            pltpu.VMEM((D, 16), jnp.float32),
            pltpu.VMEM((D, 16), jnp.float32),
            pltpu.SemaphoreType.DMA,
            pltpu.SemaphoreType.DMA,
        ],
    )
    out = gather(seqs_r, users, posItems, negItems, userRM, itemRM)
    return out.reshape(2, B)


# speedup vs baseline: 1.8114x; 1.8114x over previous
import jax
import jax.numpy as jnp
from jax import lax
from jax.experimental import pallas as pl
from jax.experimental.pallas import tpu as pltpu
from jax.experimental.pallas import tpu_sc as plsc

B = 16384
HIST = 50
D = 16
V = 1000000
NC = 2
NS = 16
NW = NC * NS
N_PER_W = B // NW
C = 64
NCHUNK = N_PER_W // C
SROWS = C * HIST
SIDX_ROWS = SROWS // 128

SB = 512
NSB = V // SB // NW * NW + 1
KMAX = 61
TAIL = V - NSB * SB


def _transpose_body(xiT, xuT, tailI, tailU, itemRM, userRM,
                    tiI, tiU, rowI, rowU, semLI, semLU, semS):
    wid = lax.axis_index("s") * NC + lax.axis_index("c")
    lane = lax.iota(jnp.int32, 16)

    def fire_loads(k, soff):
        off = pl.multiple_of((wid + k * NW) * SB, 128)
        pltpu.async_copy(xiT.at[pl.ds(0, 8), pl.ds(off, SB)],
                         tiI.at[pl.ds(0, 8), pl.ds(soff, SB)], semLI)
        pltpu.async_copy(xiT.at[pl.ds(8, 8), pl.ds(off, SB)],
                         tiI.at[pl.ds(8, 8), pl.ds(soff, SB)], semLI)
        pltpu.async_copy(xuT.at[pl.ds(0, 8), pl.ds(off, SB)],
                         tiU.at[pl.ds(0, 8), pl.ds(soff, SB)], semLU)
        pltpu.async_copy(xuT.at[pl.ds(8, 8), pl.ds(off, SB)],
                         tiU.at[pl.ds(8, 8), pl.ds(soff, SB)], semLU)

    def wait_loads(sem, ti):
        for h in (0, 8):
            pltpu.make_async_copy(xiT.at[pl.ds(0, 8), pl.ds(0, SB)],
                                  ti.at[pl.ds(h, 8), pl.ds(0, SB)], sem).wait()

    tvecs = [(lane + j) & 15 for j in range(16)]
    svecs = [t * D + lane for t in tvecs]

    def transpose_block(ti, soff, row, roff):
        @plsc.parallel_loop(0, SB // 16, unroll=2)
        def colgrp(g):
            cbase = soff + g * 16
            obase = roff + g * 256
            for j in range(16):
                vec = plsc.load_gather(ti, [lane, cbase + tvecs[j]])
                plsc.store_scatter(row, [obase + svecs[j]], vec)

    def drain_store(row, roff, dst):
        pltpu.make_async_copy(row.at[pl.ds(roff, SB * D)],
                              dst.at[pl.ds(0, SB * D)], semS).wait()

    def body(k, _):
        s = k % 2
        soff = pl.multiple_of(s * SB, 128)
        roff = pl.multiple_of(s * SB * D, 8)
        off = pl.multiple_of((wid + k * NW) * SB * D, 8)

        @pl.when(k >= 2)
        def _():
            drain_store(rowI, roff, itemRM)
            drain_store(rowU, roff, userRM)

        @pl.when(k + 1 < KMAX)
        def _():
            fire_loads(k + 1, pl.multiple_of((1 - s) * SB, 128))

        wait_loads(semLI, tiI)
        transpose_block(tiI, soff, rowI, roff)
        pltpu.async_copy(rowI.at[pl.ds(roff, SB * D)],
                         itemRM.at[pl.ds(off, SB * D)], semS)

        wait_loads(semLU, tiU)
        transpose_block(tiU, soff, rowU, roff)
        pltpu.async_copy(rowU.at[pl.ds(roff, SB * D)],
                         userRM.at[pl.ds(off, SB * D)], semS)
        return 0

    fire_loads(0, 0)
    lax.fori_loop(0, KMAX, body, 0)
    for s in (0, 1):
        drain_store(rowI, s * SB * D, itemRM)
        drain_store(rowU, s * SB * D, userRM)

    @pl.when(wid == 0)
    def _():
        off = pl.multiple_of((NSB - 1) * SB, 128)
        pltpu.sync_copy(xiT.at[pl.ds(0, 8), pl.ds(off, SB)],
                        tiI.at[pl.ds(0, 8), pl.ds(0, SB)])
        pltpu.sync_copy(xiT.at[pl.ds(8, 8), pl.ds(off, SB)],
                        tiI.at[pl.ds(8, 8), pl.ds(0, SB)])
        transpose_block(tiI, 0, rowI, 0)
        pltpu.sync_copy(rowI.at[pl.ds(0, SB * D)],
                        itemRM.at[pl.ds((NSB - 1) * SB * D, SB * D)])
        pltpu.sync_copy(tailI, itemRM.at[pl.ds(NSB * SB * D, TAIL * D)])

    @pl.when(wid == NW - 1)
    def _():
        off = pl.multiple_of((NSB - 1) * SB, 128)
        pltpu.sync_copy(xuT.at[pl.ds(0, 8), pl.ds(off, SB)],
                        tiU.at[pl.ds(0, 8), pl.ds(0, SB)])
        pltpu.sync_copy(xuT.at[pl.ds(8, 8), pl.ds(off, SB)],
                        tiU.at[pl.ds(8, 8), pl.ds(0, SB)])
        transpose_block(tiU, 0, rowU, 0)
        pltpu.sync_copy(rowU.at[pl.ds(0, SB * D)],
                        userRM.at[pl.ds((NSB - 1) * SB * D, SB * D)])
        pltpu.sync_copy(tailU, userRM.at[pl.ds(NSB * SB * D, TAIL * D)])


def _gather_body(seqs_hbm, users_hbm, pos_hbm, neg_hbm, eu_hbm, ei_hbm,
                 out_hbm, sidx, srows, uidx, pidx, nidx, urows, prows, nrows,
                 psc, nsc, pt, nt, sem0, sem1):
    wid = lax.axis_index("s") * NC + lax.axis_index("c")
    base = wid * N_PER_W
    sems = (sem0, sem1)
    lane = lax.iota(jnp.int32, 16)

    def fire(c):
        s = c % 2
        g = wid * NCHUNK + c
        pltpu.sync_copy(seqs_hbm.at[pl.ds(g * SROWS, SROWS)], sidx.at[s])
        pltpu.sync_copy(users_hbm.at[pl.ds(g * C, C)], uidx.at[s])
        pltpu.sync_copy(pos_hbm.at[pl.ds(g * C, C)], pidx.at[s])
        pltpu.sync_copy(neg_hbm.at[pl.ds(g * C, C)], nidx.at[s])
        cps = []
        for j in range(SIDX_ROWS):
            cps.append(pltpu.async_copy(
                ei_hbm.at[sidx.at[s].at[pl.ds(j * 128, 128)]],
                srows.at[s].at[pl.ds(j * 128, 128)], sems[s]))
        cps.append(pltpu.async_copy(eu_hbm.at[uidx.at[s]], urows.at[s], sems[s]))
        cps.append(pltpu.async_copy(ei_hbm.at[pidx.at[s]], prows.at[s], sems[s]))
        cps.append(pltpu.async_copy(ei_hbm.at[nidx.at[s]], nrows.at[s], sems[s]))
        return cps

    def compute(c):
        s = c % 2
        srows_s, urows_s, prows_s, nrows_s = (
            srows.at[s], urows.at[s], prows.at[s], nrows.at[s])

        def group(g, _):
            def item(l, _):
                i = g * 16 + l
                ib = i * HIST
                a0 = srows_s[ib + 0, :]
                a1 = srows_s[ib + 1, :]
                a2 = srows_s[ib + 2, :]
                a3 = srows_s[ib + 3, :]
                for j in range(4, HIST, 4):
                    a0 = a0 + srows_s[ib + j + 0, :]
                    a1 = a1 + srows_s[ib + j + 1, :]
                    if j + 2 < HIST:
                        a2 = a2 + srows_s[ib + j + 2, :]
                        a3 = a3 + srows_s[ib + j + 3, :]
                acc = (a0 + a1) + (a2 + a3)
                h = urows_s[i, :] + acc * (1.0 / HIST)
                col = jnp.full((16,), l, jnp.int32)
                plsc.store_scatter(pt, [lane, col], h * prows_s[i, :])
                plsc.store_scatter(nt, [lane, col], h * nrows_s[i, :])
                return 0

            lax.fori_loop(0, 16, item, 0)
            pvec = pt[0, :]
            nvec = nt[0, :]
            for d in range(1, D):
                pvec = pvec + pt[d, :]
                nvec = nvec + nt[d, :]
            off = (c * C) + g * 16
            psc[pl.ds(off, 16)] = pvec
            nsc[pl.ds(off, 16)] = nvec
            return 0

        lax.fori_loop(0, C // 16, group, 0)

    pending = fire(0)
    for c in range(NCHUNK):
        nxt = fire(c + 1) if c + 1 < NCHUNK else []
        for cp in pending:
            cp.wait()
        compute(c)
        pending = nxt

    pltpu.sync_copy(psc, out_hbm.at[pl.ds(base, N_PER_W)])
    pltpu.sync_copy(nsc, out_hbm.at[pl.ds(B + base, N_PER_W)])


@jax.jit
def kernel(users, seqs, posItems, negItems, embedUser, embedItem):
    seqs_r = seqs.reshape(B * HIST)
    tailI = embedItem[NSB * SB:, :].reshape(TAIL * D)
    tailU = embedUser[NSB * SB:, :].reshape(TAIL * D)

    mesh = plsc.VectorSubcoreMesh(core_axis_name="c", subcore_axis_name="s")

    transpose = pl.kernel(
        _transpose_body,
        out_type=(jax.ShapeDtypeStruct((V * D,), jnp.float32),
                  jax.ShapeDtypeStruct((V * D,), jnp.float32)),
        mesh=mesh,
        compiler_params=pltpu.CompilerParams(
            needs_layout_passes=False, use_tc_tiling_on_sc=True),
        scratch_types=[
            pltpu.VMEM((16, 2 * SB), jnp.float32),
            pltpu.VMEM((16, 2 * SB), jnp.float32),
            pltpu.VMEM((2 * SB * D,), jnp.float32),
            pltpu.VMEM((2 * SB * D,), jnp.float32),
            pltpu.SemaphoreType.DMA,
            pltpu.SemaphoreType.DMA,
            pltpu.SemaphoreType.DMA,
        ],
    )
    itemRM, userRM = transpose(embedItem.T, embedUser.T, tailI, tailU)
    itemRM = itemRM.reshape(V, D)
    userRM = userRM.reshape(V, D)

    gather = pl.kernel(
        _gather_body,
        out_type=jax.ShapeDtypeStruct((2 * B,), jnp.float32),
        mesh=mesh,
        compiler_params=pltpu.CompilerParams(
            needs_layout_passes=False, use_tc_tiling_on_sc=False),
        scratch_types=[
            pltpu.VMEM((2, SROWS), jnp.int32),
            pltpu.VMEM((2, SROWS, D), jnp.float32),
            pltpu.VMEM((2, C), jnp.int32),
            pltpu.VMEM((2, C), jnp.int32),
            pltpu.VMEM((2, C), jnp.int32),
            pltpu.VMEM((2, C, D), jnp.float32),
            pltpu.VMEM((2, C, D), jnp.float32),
            pltpu.VMEM((2, C, D), jnp.float32),
            pltpu.VMEM((N_PER_W,), jnp.float32),
            pltpu.VMEM((N_PER_W,), jnp.float32),
            pltpu.VMEM((D, 16), jnp.float32),
            pltpu.VMEM((D, 16), jnp.float32),
            pltpu.SemaphoreType.DMA,
            pltpu.SemaphoreType.DMA,
        ],
    )
    out = gather(seqs_r, users, posItems, negItems, userRM, itemRM)
    return out.reshape(2, B)


# speedup vs baseline: 1.9167x; 1.0581x over previous
import jax
import jax.numpy as jnp
from jax import lax
from jax.experimental import pallas as pl
from jax.experimental.pallas import tpu as pltpu
from jax.experimental.pallas import tpu_sc as plsc

B = 16384
HIST = 50
D = 16
V = 1000000
NC = 2
NS = 16
NW = NC * NS
N_PER_W = B // NW
C = 64
NCHUNK = N_PER_W // C
SROWS = C * HIST
SIDX_ROWS = SROWS // 128

SB = 512
NSB = V // SB // NW * NW + 1
KMAX = 61
TAIL = V - NSB * SB


def _transpose_body(xiT, xuT, tailI, tailU, itemRM, userRM,
                    tiI, tiU, rowI, rowU, semLI, semLU, semS):
    wid = lax.axis_index("s") * NC + lax.axis_index("c")
    lane = lax.iota(jnp.int32, 16)

    def fire_loads(k, soff):
        off = pl.multiple_of((wid + k * NW) * SB, 128)
        pltpu.async_copy(xiT.at[pl.ds(0, 8), pl.ds(off, SB)],
                         tiI.at[pl.ds(0, 8), pl.ds(soff, SB)], semLI)
        pltpu.async_copy(xiT.at[pl.ds(8, 8), pl.ds(off, SB)],
                         tiI.at[pl.ds(8, 8), pl.ds(soff, SB)], semLI)
        pltpu.async_copy(xuT.at[pl.ds(0, 8), pl.ds(off, SB)],
                         tiU.at[pl.ds(0, 8), pl.ds(soff, SB)], semLU)
        pltpu.async_copy(xuT.at[pl.ds(8, 8), pl.ds(off, SB)],
                         tiU.at[pl.ds(8, 8), pl.ds(soff, SB)], semLU)

    def wait_loads(sem, ti):
        for h in (0, 8):
            pltpu.make_async_copy(xiT.at[pl.ds(0, 8), pl.ds(0, SB)],
                                  ti.at[pl.ds(h, 8), pl.ds(0, SB)], sem).wait()

    tvecs = [(lane + j) & 15 for j in range(16)]
    svecs = [t * D + lane for t in tvecs]

    def transpose_block(ti, soff, row, roff):
        @plsc.parallel_loop(0, SB // 16, unroll=4)
        def colgrp(g):
            cbase = soff + g * 16
            obase = roff + g * 256
            for j in range(16):
                vec = plsc.load_gather(ti, [lane, cbase + tvecs[j]])
                plsc.store_scatter(row, [obase + svecs[j]], vec)

    def drain_store(row, roff, dst):
        pltpu.make_async_copy(row.at[pl.ds(roff, SB * D)],
                              dst.at[pl.ds(0, SB * D)], semS).wait()

    def body(k, _):
        s = k % 2
        soff = pl.multiple_of(s * SB, 128)
        roff = pl.multiple_of(s * SB * D, 8)
        off = pl.multiple_of((wid + k * NW) * SB * D, 8)

        @pl.when(k >= 2)
        def _():
            drain_store(rowI, roff, itemRM)
            drain_store(rowU, roff, userRM)

        @pl.when(k + 1 < KMAX)
        def _():
            fire_loads(k + 1, pl.multiple_of((1 - s) * SB, 128))

        wait_loads(semLI, tiI)
        transpose_block(tiI, soff, rowI, roff)
        pltpu.async_copy(rowI.at[pl.ds(roff, SB * D)],
                         itemRM.at[pl.ds(off, SB * D)], semS)

        wait_loads(semLU, tiU)
        transpose_block(tiU, soff, rowU, roff)
        pltpu.async_copy(rowU.at[pl.ds(roff, SB * D)],
                         userRM.at[pl.ds(off, SB * D)], semS)
        return 0

    fire_loads(0, 0)
    lax.fori_loop(0, KMAX, body, 0)
    for s in (0, 1):
        drain_store(rowI, s * SB * D, itemRM)
        drain_store(rowU, s * SB * D, userRM)

    @pl.when(wid == 0)
    def _():
        off = pl.multiple_of((NSB - 1) * SB, 128)
        pltpu.sync_copy(xiT.at[pl.ds(0, 8), pl.ds(off, SB)],
                        tiI.at[pl.ds(0, 8), pl.ds(0, SB)])
        pltpu.sync_copy(xiT.at[pl.ds(8, 8), pl.ds(off, SB)],
                        tiI.at[pl.ds(8, 8), pl.ds(0, SB)])
        transpose_block(tiI, 0, rowI, 0)
        pltpu.sync_copy(rowI.at[pl.ds(0, SB * D)],
                        itemRM.at[pl.ds((NSB - 1) * SB * D, SB * D)])
        pltpu.sync_copy(tailI, itemRM.at[pl.ds(NSB * SB * D, TAIL * D)])

    @pl.when(wid == NW - 1)
    def _():
        off = pl.multiple_of((NSB - 1) * SB, 128)
        pltpu.sync_copy(xuT.at[pl.ds(0, 8), pl.ds(off, SB)],
                        tiU.at[pl.ds(0, 8), pl.ds(0, SB)])
        pltpu.sync_copy(xuT.at[pl.ds(8, 8), pl.ds(off, SB)],
                        tiU.at[pl.ds(8, 8), pl.ds(0, SB)])
        transpose_block(tiU, 0, rowU, 0)
        pltpu.sync_copy(rowU.at[pl.ds(0, SB * D)],
                        userRM.at[pl.ds((NSB - 1) * SB * D, SB * D)])
        pltpu.sync_copy(tailU, userRM.at[pl.ds(NSB * SB * D, TAIL * D)])


def _gather_body(seqs_hbm, users_hbm, pos_hbm, neg_hbm, eu_hbm, ei_hbm,
                 out_hbm, sidx, srows, uidx, pidx, nidx, urows, prows, nrows,
                 psc, nsc, pt, nt, sem0, sem1):
    wid = lax.axis_index("s") * NC + lax.axis_index("c")
    base = wid * N_PER_W
    sems = (sem0, sem1)
    lane = lax.iota(jnp.int32, 16)

    def fire(c):
        s = c % 2
        g = wid * NCHUNK + c
        pltpu.sync_copy(seqs_hbm.at[pl.ds(g * SROWS, SROWS)], sidx.at[s])
        pltpu.sync_copy(users_hbm.at[pl.ds(g * C, C)], uidx.at[s])
        pltpu.sync_copy(pos_hbm.at[pl.ds(g * C, C)], pidx.at[s])
        pltpu.sync_copy(neg_hbm.at[pl.ds(g * C, C)], nidx.at[s])
        cps = []
        for j in range(SIDX_ROWS):
            cps.append(pltpu.async_copy(
                ei_hbm.at[sidx.at[s].at[pl.ds(j * 128, 128)]],
                srows.at[s].at[pl.ds(j * 128, 128)], sems[s]))
        cps.append(pltpu.async_copy(eu_hbm.at[uidx.at[s]], urows.at[s], sems[s]))
        cps.append(pltpu.async_copy(ei_hbm.at[pidx.at[s]], prows.at[s], sems[s]))
        cps.append(pltpu.async_copy(ei_hbm.at[nidx.at[s]], nrows.at[s], sems[s]))
        return cps

    def compute(c):
        s = c % 2
        srows_s, urows_s, prows_s, nrows_s = (
            srows.at[s], urows.at[s], prows.at[s], nrows.at[s])

        def group(g, _):
            @plsc.parallel_loop(0, 16, unroll=2)
            def item(l):
                i = g * 16 + l
                ib = i * HIST
                a0 = srows_s[ib + 0, :]
                a1 = srows_s[ib + 1, :]
                a2 = srows_s[ib + 2, :]
                a3 = srows_s[ib + 3, :]
                for j in range(4, HIST, 4):
                    a0 = a0 + srows_s[ib + j + 0, :]
                    a1 = a1 + srows_s[ib + j + 1, :]
                    if j + 2 < HIST:
                        a2 = a2 + srows_s[ib + j + 2, :]
                        a3 = a3 + srows_s[ib + j + 3, :]
                acc = (a0 + a1) + (a2 + a3)
                h = urows_s[i, :] + acc * (1.0 / HIST)
                col = jnp.full((16,), l, jnp.int32)
                plsc.store_scatter(pt, [lane, col], h * prows_s[i, :])
                plsc.store_scatter(nt, [lane, col], h * nrows_s[i, :])

            pvec = pt[0, :]
            nvec = nt[0, :]
            for d in range(1, D):
                pvec = pvec + pt[d, :]
                nvec = nvec + nt[d, :]
            off = (c * C) + g * 16
            psc[pl.ds(off, 16)] = pvec
            nsc[pl.ds(off, 16)] = nvec
            return 0

        lax.fori_loop(0, C // 16, group, 0)

    pending = fire(0)
    for c in range(NCHUNK):
        nxt = fire(c + 1) if c + 1 < NCHUNK else []
        for cp in pending:
            cp.wait()
        compute(c)
        pending = nxt

    pltpu.sync_copy(psc, out_hbm.at[pl.ds(base, N_PER_W)])
    pltpu.sync_copy(nsc, out_hbm.at[pl.ds(B + base, N_PER_W)])


@jax.jit
def kernel(users, seqs, posItems, negItems, embedUser, embedItem):
    seqs_r = seqs.reshape(B * HIST)
    tailI = embedItem[NSB * SB:, :].reshape(TAIL * D)
    tailU = embedUser[NSB * SB:, :].reshape(TAIL * D)

    mesh = plsc.VectorSubcoreMesh(core_axis_name="c", subcore_axis_name="s")

    transpose = pl.kernel(
        _transpose_body,
        out_type=(jax.ShapeDtypeStruct((V * D,), jnp.float32),
                  jax.ShapeDtypeStruct((V * D,), jnp.float32)),
        mesh=mesh,
        compiler_params=pltpu.CompilerParams(
            needs_layout_passes=False, use_tc_tiling_on_sc=True),
        scratch_types=[
            pltpu.VMEM((16, 2 * SB), jnp.float32),
            pltpu.VMEM((16, 2 * SB), jnp.float32),
            pltpu.VMEM((2 * SB * D,), jnp.float32),
            pltpu.VMEM((2 * SB * D,), jnp.float32),
            pltpu.SemaphoreType.DMA,
            pltpu.SemaphoreType.DMA,
            pltpu.SemaphoreType.DMA,
        ],
    )
    itemRM, userRM = transpose(embedItem.T, embedUser.T, tailI, tailU)
    itemRM = itemRM.reshape(V, D)
    userRM = userRM.reshape(V, D)

    gather = pl.kernel(
        _gather_body,
        out_type=jax.ShapeDtypeStruct((2 * B,), jnp.float32),
        mesh=mesh,
        compiler_params=pltpu.CompilerParams(
            needs_layout_passes=False, use_tc_tiling_on_sc=False),
        scratch_types=[
            pltpu.VMEM((2, SROWS), jnp.int32),
            pltpu.VMEM((2, SROWS, D), jnp.float32),
            pltpu.VMEM((2, C), jnp.int32),
            pltpu.VMEM((2, C), jnp.int32),
            pltpu.VMEM((2, C), jnp.int32),
            pltpu.VMEM((2, C, D), jnp.float32),
            pltpu.VMEM((2, C, D), jnp.float32),
            pltpu.VMEM((2, C, D), jnp.float32),
            pltpu.VMEM((N_PER_W,), jnp.float32),
            pltpu.VMEM((N_PER_W,), jnp.float32),
            pltpu.VMEM((D, 16), jnp.float32),
            pltpu.VMEM((D, 16), jnp.float32),
            pltpu.SemaphoreType.DMA,
            pltpu.SemaphoreType.DMA,
        ],
    )
    out = gather(seqs_r, users, posItems, negItems, userRM, itemRM)
    return out.reshape(2, B)
